# baseline (speedup 1.0000x reference)
import functools

import jax
import jax.numpy as jnp
from jax import lax
from jax.experimental import pallas as pl
from jax.experimental.pallas import tpu as pltpu
from jax.experimental.pallas import tpu_sc as plsc

_D = 32
_B = 16384

_INFO = plsc.get_sparse_core_info()
_NW = _INFO.num_cores * _INFO.num_subcores
_B_PER_W = _B // _NW
_MB = 8

_MESH = plsc.VectorSubcoreMesh(core_axis_name="c", subcore_axis_name="s")


@functools.partial(
    pl.kernel,
    mesh=_MESH,
    out_type=jax.ShapeDtypeStruct((_D, _B), jnp.float32),
    scratch_types=[
        pltpu.VMEM((_B_PER_W,), jnp.int32),
        pltpu.VMEM((_MB, _D, 128), jnp.float32),
        pltpu.VMEM((_MB, _D, 128), jnp.float32),
        pltpu.VMEM((_D, _B_PER_W), jnp.float32),
        pltpu.SemaphoreType.DMA,
        pltpu.SemaphoreType.DMA,
    ],
    compiler_params=pltpu.CompilerParams(needs_layout_passes=False),
)
def _gather_kernel(
    table_hbm, idx_hbm, out_hbm, idx_v, sb_a, sb_b, rows_v, sem_a, sem_b
):
    wid = lax.axis_index("s") * _INFO.num_cores + lax.axis_index("c")
    base = wid * _B_PER_W
    pltpu.sync_copy(idx_hbm.at[pl.ds(base, _B_PER_W)], idx_v)
    iota16 = lax.iota(jnp.int32, 16)

    def fire(vtile, lane0, sb, sem):
        for k in range(_MB):
            pltpu.async_copy(
                table_hbm.at[:, pl.ds(vtile[lane0 + k] * 128, 128)],
                sb.at[k],
                sem,
            )

    def consume(vlane, lane0, col0, sb, sem):
        for k in range(_MB):
            pltpu.make_async_copy(
                table_hbm.at[:, pl.ds(0, 128)], sb.at[k], sem
            ).wait()
            lane16 = jnp.full((16,), vlane[lane0 + k], jnp.int32)
            col16 = jnp.full((16,), col0 + k, jnp.int32)
            for half in range(2):
                rsel = iota16 + jnp.int32(half * 16)
                vals = plsc.load_gather(sb.at[k], [rsel, lane16])
                plsc.store_scatter(rows_v, [rsel, col16], vals)

    v0 = idx_v[pl.ds(0, 16)]
    fire(v0 >> jnp.int32(7), 0, sb_a, sem_a)

    def body(i, _):
        jb = i * 16
        va = idx_v[pl.ds(jb, 16)]
        vtile_a = va >> jnp.int32(7)
        vlane_a = va & jnp.int32(127)
        fire(vtile_a, _MB, sb_b, sem_b)
        consume(vlane_a, 0, jb, sb_a, sem_a)
        vb = idx_v[pl.ds((jb + 16) & jnp.int32(_B_PER_W - 1), 16)]
        fire(vb >> jnp.int32(7), 0, sb_a, sem_a)
        consume(vlane_a, _MB, jb + _MB, sb_b, sem_b)
        return 0

    lax.fori_loop(0, _B_PER_W // 16, body, 0)
    for k in range(_MB):
        pltpu.make_async_copy(
            table_hbm.at[:, pl.ds(0, 128)], sb_a.at[k], sem_a
        ).wait()
    pltpu.sync_copy(rows_v, out_hbm.at[:, pl.ds(base, _B_PER_W)])


def kernel(instance_ids, embedding_instance_weight):
    ids = instance_ids.astype(jnp.int32)
    tab_t = embedding_instance_weight.T
    out_t = _gather_kernel(tab_t, ids)
    return out_t.T


# speedup vs baseline: 1.0043x; 1.0043x over previous
import functools

import jax
import jax.numpy as jnp
from jax import lax
from jax.experimental import pallas as pl
from jax.experimental.pallas import tpu as pltpu
from jax.experimental.pallas import tpu_sc as plsc

_D = 32
_B = 16384

_INFO = plsc.get_sparse_core_info()
_NW = _INFO.num_cores * _INFO.num_subcores
_B_PER_W = _B // _NW
_MB = 16

_MESH = plsc.VectorSubcoreMesh(core_axis_name="c", subcore_axis_name="s")


@functools.partial(
    pl.kernel,
    mesh=_MESH,
    out_type=jax.ShapeDtypeStruct((_D, _B), jnp.float32),
    scratch_types=[
        pltpu.VMEM((_B_PER_W,), jnp.int32),
        pltpu.VMEM((_MB, _D, 128), jnp.float32),
        pltpu.VMEM((_D, _B_PER_W), jnp.float32),
        pltpu.SemaphoreType.DMA,
    ],
    compiler_params=pltpu.CompilerParams(needs_layout_passes=False),
)
def _gather_kernel(table_hbm, idx_hbm, out_hbm, idx_v, sb_v, rows_v, sem):
    wid = lax.axis_index("s") * _INFO.num_cores + lax.axis_index("c")
    base = wid * _B_PER_W
    pltpu.sync_copy(idx_hbm.at[pl.ds(base, _B_PER_W)], idx_v)
    iota16 = lax.iota(jnp.int32, 16)

    def group(g, _):
        jbase = g * _MB
        v16 = idx_v[pl.ds(jbase, _MB)]
        vtile = v16 >> jnp.int32(7)
        vlane = v16 & jnp.int32(127)
        copies = []
        for k in range(_MB):
            copies.append(
                pltpu.async_copy(
                    table_hbm.at[:, pl.ds(vtile[k] * 128, 128)],
                    sb_v.at[k],
                    sem,
                )
            )
        for k in range(_MB):
            copies[k].wait()
            lane16 = jnp.full((16,), vlane[k], jnp.int32)
            col16 = jnp.full((16,), jbase + k, jnp.int32)
            for half in range(2):
                rsel = iota16 + jnp.int32(half * 16)
                vals = plsc.load_gather(sb_v.at[k], [rsel, lane16])
                plsc.store_scatter(rows_v, [rsel, col16], vals)
        return 0

    lax.fori_loop(0, _B_PER_W // _MB, group, 0)
    pltpu.sync_copy(rows_v, out_hbm.at[:, pl.ds(base, _B_PER_W)])


def kernel(instance_ids, embedding_instance_weight):
    ids = instance_ids.astype(jnp.int32)
    tab_t = embedding_instance_weight.T
    out_t = _gather_kernel(tab_t, ids)
    return out_t.T
